# baseline (speedup 1.0000x reference)
import functools

import jax
import jax.numpy as jnp
from jax import lax
from jax.experimental import pallas as pl
from jax.experimental.pallas import tpu as pltpu
from jax.experimental.pallas import tpu_sc as plsc

_LANES = 16
_ROW = 128


def _emb_kernel(B, L, D):
    NC, NS = 2, 16
    NW = NC * NS
    seq_per_w = B // NW

    mesh = plsc.VectorSubcoreMesh(core_axis_name="c", subcore_axis_name="s")

    @functools.partial(
        pl.kernel,
        mesh=mesh,
        compiler_params=pltpu.CompilerParams(use_tc_tiling_on_sc=False),
        out_type=jax.ShapeDtypeStruct((B, L, D), jnp.float32),
        scratch_types=[
            pltpu.VMEM((seq_per_w, L), jnp.int32),
            pltpu.VMEM((L, D), jnp.float32),
            pltpu.VMEM((L, _ROW), jnp.float32),
            pltpu.VMEM((L, _ROW), jnp.float32),
            pltpu.VMEM((L, D), jnp.float32),
            pltpu.VMEM((L, D), jnp.float32),
            pltpu.SemaphoreType.DMA,
            pltpu.SemaphoreType.DMA,
            pltpu.SemaphoreType.DMA,
            pltpu.SemaphoreType.DMA,
        ],
    )
    def k(x_hbm, tok_hbm, pos_hbm, out_hbm, idx_v, pos_v,
          gb0, gb1, cb0, cb1, g0, g1, s0, s1):
        gbufs = (gb0, gb1)
        cbufs = (cb0, cb1)
        gsems = (g0, g1)
        ssems = (s0, s1)
        wid = lax.axis_index("s") * NC + lax.axis_index("c")
        seq0 = wid * seq_per_w
        pltpu.sync_copy(x_hbm.at[pl.ds(seq0, seq_per_w), :], idx_v)
        pltpu.sync_copy(pos_hbm.at[pl.ds(0, L), :], pos_v)

        pltpu.async_copy(tok_hbm.at[idx_v.at[0]], gbufs[0], gsems[0])

        def body(g, carry):
            for b in range(2):
                s = 2 * g + b
                gbuf, cbuf = gbufs[b], cbufs[b]
                pltpu.make_async_copy(
                    tok_hbm.at[idx_v.at[0]], gbuf, gsems[b]
                ).wait()

                @pl.when(s + 1 < seq_per_w)
                def _issue():
                    pltpu.async_copy(
                        tok_hbm.at[idx_v.at[s + 1]], gbufs[1 - b],
                        gsems[1 - b],
                    )

                @pl.when(s >= 2)
                def _drain():
                    pltpu.make_async_copy(
                        cbuf, out_hbm.at[seq0], ssems[b]
                    ).wait()

                @plsc.parallel_loop(0, L, unroll=8)
                def _add(r):
                    for c in range(D // _LANES):
                        sl = pl.ds(c * _LANES, _LANES)
                        cbuf[r, sl] = gbuf[r, sl] + pos_v[r, sl]

                pltpu.async_copy(cbuf, out_hbm.at[seq0 + s], ssems[b])

            return carry

        lax.fori_loop(0, seq_per_w // 2, body, 0)

        for b in range(2):
            pltpu.make_async_copy(
                cbufs[b], out_hbm.at[seq0], ssems[b]
            ).wait()

    return k


def kernel(x, token_table, pos_table):
    B, L = x.shape
    D = token_table.shape[1]
    tpad = jnp.pad(token_table, ((0, 0), (0, _ROW - D)))
    out = _emb_kernel(B, L, D)(x.astype(jnp.int32), tpad, pos_table)
    return out


# speedup vs baseline: 1.0121x; 1.0121x over previous
import functools

import jax
import jax.numpy as jnp
from jax import lax
from jax.experimental import pallas as pl
from jax.experimental.pallas import tpu as pltpu
from jax.experimental.pallas import tpu_sc as plsc

_LANES = 16


def _emb_kernel(B, L, D):
    NC, NS = 2, 16
    NW = NC * NS
    seq_per_w = B // NW

    mesh = plsc.VectorSubcoreMesh(core_axis_name="c", subcore_axis_name="s")

    @functools.partial(
        pl.kernel,
        mesh=mesh,
        compiler_params=pltpu.CompilerParams(use_tc_tiling_on_sc=False),
        out_type=jax.ShapeDtypeStruct((B, L, D), jnp.float32),
        scratch_types=[
            pltpu.VMEM((seq_per_w, L), jnp.int32),
            pltpu.VMEM((L, D), jnp.float32),
            pltpu.VMEM((L, D), jnp.float32),
            pltpu.VMEM((L, D), jnp.float32),
            pltpu.VMEM((L, D), jnp.float32),
            pltpu.VMEM((L, D), jnp.float32),
            pltpu.SemaphoreType.DMA,
            pltpu.SemaphoreType.DMA,
            pltpu.SemaphoreType.DMA,
            pltpu.SemaphoreType.DMA,
        ],
    )
    def k(x_hbm, tok_hbm, pos_hbm, out_hbm, idx_v, pos_v,
          gb0, gb1, cb0, cb1, g0, g1, s0, s1):
        gbufs = (gb0, gb1)
        cbufs = (cb0, cb1)
        gsems = (g0, g1)
        ssems = (s0, s1)
        wid = lax.axis_index("s") * NC + lax.axis_index("c")
        seq0 = wid * seq_per_w
        pltpu.sync_copy(x_hbm.at[pl.ds(seq0, seq_per_w), :], idx_v)
        pltpu.sync_copy(pos_hbm.at[pl.ds(0, L), :], pos_v)

        pltpu.async_copy(tok_hbm.at[idx_v.at[0]], gbufs[0], gsems[0])

        def body(g, carry):
            for b in range(2):
                s = 2 * g + b
                gbuf, cbuf = gbufs[b], cbufs[b]
                pltpu.make_async_copy(
                    tok_hbm.at[idx_v.at[0]], gbuf, gsems[b]
                ).wait()

                @pl.when(s + 1 < seq_per_w)
                def _issue():
                    pltpu.async_copy(
                        tok_hbm.at[idx_v.at[s + 1]], gbufs[1 - b],
                        gsems[1 - b],
                    )

                @pl.when(s >= 2)
                def _drain():
                    pltpu.make_async_copy(
                        cbuf, out_hbm.at[seq0], ssems[b]
                    ).wait()

                @plsc.parallel_loop(0, L, unroll=8)
                def _add(r):
                    for c in range(D // _LANES):
                        sl = pl.ds(c * _LANES, _LANES)
                        cbuf[r, sl] = gbuf[r, sl] + pos_v[r, sl]

                pltpu.async_copy(cbuf, out_hbm.at[seq0 + s], ssems[b])

            return carry

        lax.fori_loop(0, seq_per_w // 2, body, 0)

        for b in range(2):
            pltpu.make_async_copy(
                cbufs[b], out_hbm.at[seq0], ssems[b]
            ).wait()

    return k


def kernel(x, token_table, pos_table):
    B, L = x.shape
    D = token_table.shape[1]
    out = _emb_kernel(B, L, D)(x.astype(jnp.int32), token_table, pos_table)
    return out


# speedup vs baseline: 1.0415x; 1.0290x over previous
import functools

import jax
import jax.numpy as jnp
from jax import lax
from jax.experimental import pallas as pl
from jax.experimental.pallas import tpu as pltpu
from jax.experimental.pallas import tpu_sc as plsc

_LANES = 16
_NBUF = 4


def _emb_kernel(B, L, D):
    NC, NS = 2, 16
    NW = NC * NS
    seq_per_w = B // NW

    mesh = plsc.VectorSubcoreMesh(core_axis_name="c", subcore_axis_name="s")

    @functools.partial(
        pl.kernel,
        mesh=mesh,
        compiler_params=pltpu.CompilerParams(use_tc_tiling_on_sc=False),
        out_type=jax.ShapeDtypeStruct((B, L, D), jnp.float32),
        scratch_types=(
            [
                pltpu.VMEM((seq_per_w, L), jnp.int32),
                pltpu.VMEM((L, D), jnp.float32),
            ]
            + [pltpu.VMEM((L, D), jnp.float32)] * _NBUF
            + [pltpu.SemaphoreType.DMA] * (2 * _NBUF)
        ),
    )
    def k(x_hbm, tok_hbm, pos_hbm, out_hbm, idx_v, pos_v, *bufs_and_sems):
        rows = bufs_and_sems[:_NBUF]
        gsems = bufs_and_sems[_NBUF:2 * _NBUF]
        ssems = bufs_and_sems[2 * _NBUF:]
        wid = lax.axis_index("s") * NC + lax.axis_index("c")
        seq0 = wid * seq_per_w
        pltpu.sync_copy(x_hbm.at[pl.ds(seq0, seq_per_w), :], idx_v)
        pltpu.sync_copy(pos_hbm.at[pl.ds(0, L), :], pos_v)

        for b in range(_NBUF - 1):
            pltpu.async_copy(tok_hbm.at[idx_v.at[b]], rows[b], gsems[b])

        def body(g, carry):
            for b in range(_NBUF):
                s = g * _NBUF + b
                buf = rows[b]
                pltpu.make_async_copy(
                    tok_hbm.at[idx_v.at[0]], buf, gsems[b]
                ).wait()

                @plsc.parallel_loop(0, L, unroll=8)
                def _add(r):
                    for c in range(D // _LANES):
                        sl = pl.ds(c * _LANES, _LANES)
                        buf[r, sl] = buf[r, sl] + pos_v[r, sl]

                pltpu.async_copy(buf, out_hbm.at[seq0 + s], ssems[b])

                nb = (b + _NBUF - 1) % _NBUF
                ns = s + _NBUF - 1

                @pl.when(ns < seq_per_w)
                def _issue():
                    @pl.when(s >= 1)
                    def _drain():
                        pltpu.make_async_copy(
                            rows[nb], out_hbm.at[seq0], ssems[nb]
                        ).wait()

                    pltpu.async_copy(
                        tok_hbm.at[idx_v.at[ns]], rows[nb], gsems[nb]
                    )

            return carry

        lax.fori_loop(0, seq_per_w // _NBUF, body, 0)

        for b in range(_NBUF):
            pltpu.make_async_copy(
                rows[b], out_hbm.at[seq0], ssems[b]
            ).wait()

    return k


def kernel(x, token_table, pos_table):
    B, L = x.shape
    D = token_table.shape[1]
    out = _emb_kernel(B, L, D)(x.astype(jnp.int32), token_table, pos_table)
    return out


# speedup vs baseline: 1.0428x; 1.0013x over previous
import functools

import jax
import jax.numpy as jnp
from jax import lax
from jax.experimental import pallas as pl
from jax.experimental.pallas import tpu as pltpu
from jax.experimental.pallas import tpu_sc as plsc

_LANES = 16
_NBUF = 4


def _emb_kernel(B, L, D):
    NC, NS = 2, 16
    NW = NC * NS
    seq_per_w = B // NW

    mesh = plsc.VectorSubcoreMesh(core_axis_name="c", subcore_axis_name="s")

    @functools.partial(
        pl.kernel,
        mesh=mesh,
        compiler_params=pltpu.CompilerParams(use_tc_tiling_on_sc=False, allow_input_fusion=[1]),
        out_type=jax.ShapeDtypeStruct((B, L, D), jnp.float32),
        scratch_types=(
            [
                pltpu.VMEM((seq_per_w, L), jnp.int32),
                pltpu.VMEM((L, D), jnp.float32),
            ]
            + [pltpu.VMEM((L, D), jnp.float32)] * _NBUF
            + [pltpu.SemaphoreType.DMA] * (2 * _NBUF)
        ),
    )
    def k(x_hbm, tok_hbm, pos_hbm, out_hbm, idx_v, pos_v, *bufs_and_sems):
        rows = bufs_and_sems[:_NBUF]
        gsems = bufs_and_sems[_NBUF:2 * _NBUF]
        ssems = bufs_and_sems[2 * _NBUF:]
        wid = lax.axis_index("s") * NC + lax.axis_index("c")
        seq0 = wid * seq_per_w
        pltpu.sync_copy(x_hbm.at[pl.ds(seq0, seq_per_w), :], idx_v)
        pltpu.sync_copy(pos_hbm.at[pl.ds(0, L), :], pos_v)

        for b in range(_NBUF - 1):
            pltpu.async_copy(tok_hbm.at[idx_v.at[b]], rows[b], gsems[b])

        def body(g, carry):
            for b in range(_NBUF):
                s = g * _NBUF + b
                buf = rows[b]
                pltpu.make_async_copy(
                    tok_hbm.at[idx_v.at[0]], buf, gsems[b]
                ).wait()

                @plsc.parallel_loop(0, L, unroll=8)
                def _add(r):
                    for c in range(D // _LANES):
                        sl = pl.ds(c * _LANES, _LANES)
                        buf[r, sl] = buf[r, sl] + pos_v[r, sl]

                pltpu.async_copy(buf, out_hbm.at[seq0 + s], ssems[b])

                nb = (b + _NBUF - 1) % _NBUF
                ns = s + _NBUF - 1

                @pl.when(ns < seq_per_w)
                def _issue():
                    @pl.when(s >= 1)
                    def _drain():
                        pltpu.make_async_copy(
                            rows[nb], out_hbm.at[seq0], ssems[nb]
                        ).wait()

                    pltpu.async_copy(
                        tok_hbm.at[idx_v.at[ns]], rows[nb], gsems[nb]
                    )

            return carry

        lax.fori_loop(0, seq_per_w // _NBUF, body, 0)

        for b in range(_NBUF):
            pltpu.make_async_copy(
                rows[b], out_hbm.at[seq0], ssems[b]
            ).wait()

    return k


def kernel(x, token_table, pos_table):
    B, L = x.shape
    D = token_table.shape[1]
    out = _emb_kernel(B, L, D)(x.astype(jnp.int32), token_table, pos_table)
    return out
